# Initial kernel scaffold; baseline (speedup 1.0000x reference)
import jax
import jax.numpy as jnp
from jax.experimental import pallas as pl


def kernel(input, meshes, W1, b1, W2, b2):
    raise NotImplementedError("write your pallas kernel here")


# speedup vs baseline: 2.8548x; 2.8548x over previous
import functools

import jax
import jax.numpy as jnp
from jax import lax
from jax.experimental import pallas as pl
from jax.experimental.pallas import tpu as pltpu
from jax.experimental.pallas import tpu_sc as plsc

E = 160000
C = 256
_NW = 32
_R = 80
_BE = 640
_BT = 640

def _sc_gather_body(table_hbm, idx_hbm, out_hbm, idx_v, rows_v, sem):
    wid = lax.axis_index("s") * 2 + lax.axis_index("c")
    nrows = (4 * E) // _NW
    base0 = wid * nrows

    def body(i, carry):
        base = base0 + i * _R
        pltpu.sync_copy(idx_hbm.at[pl.ds(base, _R)], idx_v)
        pltpu.async_copy(table_hbm.at[idx_v], rows_v, sem).wait()
        pltpu.sync_copy(rows_v, out_hbm.at[pl.ds(base, _R)])
        return carry

    lax.fori_loop(0, nrows // _R, body, 0)


@functools.lru_cache(maxsize=1)
def _sc_gather_kernel():
    mesh = plsc.VectorSubcoreMesh(core_axis_name="c", subcore_axis_name="s")
    return pl.kernel(
        _sc_gather_body,
        mesh=mesh,
        out_type=jax.ShapeDtypeStruct((4 * E, C), jnp.float32),
        scratch_types=[
            pltpu.VMEM((_R,), jnp.int32),
            pltpu.VMEM((_R, C), jnp.float32),
            pltpu.SemaphoreType.DMA,
        ],
    )


def _sc_gather(table, idxf):
    return _sc_gather_kernel()(table, idxf)


def _tr_body(x_ref, o_ref):
    o_ref[:] = x_ref[:].T


def _transpose_tc(x2d):
    return pl.pallas_call(
        _tr_body,
        grid=(E // _BT,),
        in_specs=[pl.BlockSpec((C, _BT), lambda i: (0, i))],
        out_specs=pl.BlockSpec((_BT, C), lambda i: (i, 0)),
        out_shape=jax.ShapeDtypeStruct((E, C), jnp.float32),
    )(x2d)


def _mm_body(x_ref, nb_ref, w_ref, b_ref, o_ref, *, transpose_out):
    f0 = x_ref[:]
    n0, n1, n2, n3 = nb_ref[0], nb_ref[1], nb_ref[2], nb_ref[3]
    acc = jnp.dot(f0, w_ref[0], preferred_element_type=jnp.float32)
    acc = acc + jnp.dot(n0 + n2, w_ref[1], preferred_element_type=jnp.float32)
    acc = acc + jnp.dot(n1 + n3, w_ref[2], preferred_element_type=jnp.float32)
    acc = acc + jnp.dot(jnp.abs(n0 - n2), w_ref[3],
                        preferred_element_type=jnp.float32)
    acc = acc + jnp.dot(jnp.abs(n1 - n3), w_ref[4],
                        preferred_element_type=jnp.float32)
    acc = acc + b_ref[:]
    o_ref[:] = acc.T if transpose_out else acc


def _mesh_conv_tc(xt, nb, wt, b2d, transpose_out):
    if transpose_out:
        out_shape = jax.ShapeDtypeStruct((C, E), jnp.float32)
        out_spec = pl.BlockSpec((C, _BE), lambda i: (0, i))
    else:
        out_shape = jax.ShapeDtypeStruct((E, C), jnp.float32)
        out_spec = pl.BlockSpec((_BE, C), lambda i: (i, 0))
    return pl.pallas_call(
        functools.partial(_mm_body, transpose_out=transpose_out),
        grid=(E // _BE,),
        in_specs=[
            pl.BlockSpec((_BE, C), lambda i: (i, 0)),
            pl.BlockSpec((4, _BE, C), lambda i: (0, i, 0)),
            pl.BlockSpec((5, C, C), lambda i: (0, 0, 0)),
            pl.BlockSpec((1, C), lambda i: (0, 0)),
        ],
        out_specs=out_spec,
        out_shape=out_shape,
    )(xt, nb.reshape(4, E, C), wt, b2d)


def kernel(input, meshes, W1, b1, W2, b2):
    x2d = input.reshape(C, E)
    idxf = meshes.reshape(E, 4).astype(jnp.int32).T.reshape(4 * E)
    wt1 = jnp.transpose(W1, (2, 1, 0))
    wt2 = jnp.transpose(W2, (2, 1, 0))
    xt = _transpose_tc(x2d)
    nb1 = _sc_gather(xt, idxf)
    h1 = _mesh_conv_tc(xt, nb1, wt1, b1.reshape(1, C), False)
    nb2 = _sc_gather(h1, idxf)
    out_t = _mesh_conv_tc(h1, nb2, wt2, b2.reshape(1, C), True)
    return out_t.reshape(1, C, E)


# speedup vs baseline: 3.7657x; 1.3191x over previous
import functools

import jax
import jax.numpy as jnp
from jax import lax
from jax.experimental import pallas as pl
from jax.experimental.pallas import tpu as pltpu
from jax.experimental.pallas import tpu_sc as plsc

E = 160000
C = 256
_NW = 32
_R = 80
_BE = 640
_BT = 640

_NCH = (4 * E) // _NW // _R


def _sc_gather_body(table_hbm, idx_hbm, out_hbm,
                    idx_all, rows0, rows1, g0, g1, s0, s1):
    wid = lax.axis_index("s") * 2 + lax.axis_index("c")
    nrows = (4 * E) // _NW
    base0 = wid * nrows
    pltpu.sync_copy(idx_hbm.at[wid], idx_all)

    rows = (rows0, rows1)
    gs = (g0, g1)
    ss = (s0, s1)

    def g_start(ci, b):
        pltpu.async_copy(table_hbm.at[idx_all.at[ci]], rows[b], gs[b])

    def g_wait(b):
        pltpu.make_async_copy(table_hbm.at[idx_all.at[0]], rows[b],
                              gs[b]).wait()

    def s_start(ci, b):
        pltpu.async_copy(rows[b], out_hbm.at[pl.ds(base0 + ci * _R, _R)],
                         ss[b])

    def s_wait(b):
        pltpu.make_async_copy(rows[b], out_hbm.at[pl.ds(base0, _R)],
                              ss[b]).wait()

    g_start(0, 0)
    g_start(1, 1)
    g_wait(0)
    s_start(0, 0)

    def body(j, carry):
        c0 = 2 * j + 2
        s_wait(0)
        g_start(c0, 0)
        g_wait(1)
        s_start(c0 - 1, 1)
        s_wait(1)
        g_start(c0 + 1, 1)
        g_wait(0)
        s_start(c0, 0)
        return carry

    lax.fori_loop(0, (_NCH - 2) // 2, body, 0)
    g_wait(1)
    s_start(_NCH - 1, 1)
    s_wait(0)
    s_wait(1)


@functools.lru_cache(maxsize=1)
def _sc_gather_kernel():
    mesh = plsc.VectorSubcoreMesh(core_axis_name="c", subcore_axis_name="s")
    return pl.kernel(
        _sc_gather_body,
        mesh=mesh,
        out_type=jax.ShapeDtypeStruct((4 * E, C), jnp.float32),
        scratch_types=[
            pltpu.VMEM((_NCH, _R), jnp.int32),
            pltpu.VMEM((_R, C), jnp.float32),
            pltpu.VMEM((_R, C), jnp.float32),
            pltpu.SemaphoreType.DMA,
            pltpu.SemaphoreType.DMA,
            pltpu.SemaphoreType.DMA,
            pltpu.SemaphoreType.DMA,
        ],
    )


def _sc_gather(table, idxf):
    return _sc_gather_kernel()(table, idxf.reshape(_NW, _NCH, _R))


def _tr_body(x_ref, o_ref):
    o_ref[:] = x_ref[:].T


def _transpose_tc(x2d):
    return pl.pallas_call(
        _tr_body,
        grid=(E // _BT,),
        in_specs=[pl.BlockSpec((C, _BT), lambda i: (0, i))],
        out_specs=pl.BlockSpec((_BT, C), lambda i: (i, 0)),
        out_shape=jax.ShapeDtypeStruct((E, C), jnp.float32),
    )(x2d)


def _mm_body(x_ref, nb_ref, w_ref, b_ref, o_ref, *, transpose_out):
    f0 = x_ref[:]
    n0, n1, n2, n3 = nb_ref[0], nb_ref[1], nb_ref[2], nb_ref[3]
    acc = jnp.dot(f0, w_ref[0], preferred_element_type=jnp.float32)
    acc = acc + jnp.dot(n0 + n2, w_ref[1], preferred_element_type=jnp.float32)
    acc = acc + jnp.dot(n1 + n3, w_ref[2], preferred_element_type=jnp.float32)
    acc = acc + jnp.dot(jnp.abs(n0 - n2), w_ref[3],
                        preferred_element_type=jnp.float32)
    acc = acc + jnp.dot(jnp.abs(n1 - n3), w_ref[4],
                        preferred_element_type=jnp.float32)
    acc = acc + b_ref[:]
    o_ref[:] = acc.T if transpose_out else acc


def _mesh_conv_tc(xt, nb, wt, b2d, transpose_out):
    if transpose_out:
        out_shape = jax.ShapeDtypeStruct((C, E), jnp.float32)
        out_spec = pl.BlockSpec((C, _BE), lambda i: (0, i))
    else:
        out_shape = jax.ShapeDtypeStruct((E, C), jnp.float32)
        out_spec = pl.BlockSpec((_BE, C), lambda i: (i, 0))
    return pl.pallas_call(
        functools.partial(_mm_body, transpose_out=transpose_out),
        grid=(E // _BE,),
        in_specs=[
            pl.BlockSpec((_BE, C), lambda i: (i, 0)),
            pl.BlockSpec((4, _BE, C), lambda i: (0, i, 0)),
            pl.BlockSpec((5, C, C), lambda i: (0, 0, 0)),
            pl.BlockSpec((1, C), lambda i: (0, 0)),
        ],
        out_specs=out_spec,
        out_shape=out_shape,
    )(xt, nb.reshape(4, E, C), wt, b2d)


def kernel(input, meshes, W1, b1, W2, b2):
    x2d = input.reshape(C, E)
    idxf = meshes.reshape(E, 4).astype(jnp.int32).T.reshape(4 * E)
    wt1 = jnp.transpose(W1, (2, 1, 0))
    wt2 = jnp.transpose(W2, (2, 1, 0))
    xt = _transpose_tc(x2d)
    nb1 = _sc_gather(xt, idxf)
    h1 = _mesh_conv_tc(xt, nb1, wt1, b1.reshape(1, C), False)
    nb2 = _sc_gather(h1, idxf)
    out_t = _mesh_conv_tc(h1, nb2, wt2, b2.reshape(1, C), True)
    return out_t.reshape(1, C, E)


# speedup vs baseline: 5.0335x; 1.3366x over previous
import functools

import jax
import jax.numpy as jnp
from jax import lax
from jax.experimental import pallas as pl
from jax.experimental.pallas import tpu as pltpu
from jax.experimental.pallas import tpu_sc as plsc

E = 160000
C = 256
_NW = 32
_R = 80
_BE = 640
_BT = 640

_NCH = (4 * E) // _NW // _R


def _sc_gather_body(table_hbm, idx_hbm, out_hbm,
                    idx_all, rows0, rows1, g0, g1, s0, s1):
    wid = lax.axis_index("s") * 2 + lax.axis_index("c")
    nrows = (4 * E) // _NW
    base0 = wid * nrows
    pltpu.sync_copy(idx_hbm.at[wid], idx_all)

    rows = (rows0, rows1)
    gs = (g0, g1)
    ss = (s0, s1)

    def g_start(ci, b):
        pltpu.async_copy(table_hbm.at[idx_all.at[ci]], rows[b], gs[b])

    def g_wait(b):
        pltpu.make_async_copy(table_hbm.at[idx_all.at[0]], rows[b],
                              gs[b]).wait()

    def s_start(ci, b):
        pltpu.async_copy(rows[b], out_hbm.at[pl.ds(base0 + ci * _R, _R)],
                         ss[b])

    def s_wait(b):
        pltpu.make_async_copy(rows[b], out_hbm.at[pl.ds(base0, _R)],
                              ss[b]).wait()

    g_start(0, 0)
    g_start(1, 1)
    g_wait(0)
    s_start(0, 0)

    def body(j, carry):
        c0 = 2 * j + 2
        s_wait(0)
        g_start(c0, 0)
        g_wait(1)
        s_start(c0 - 1, 1)
        s_wait(1)
        g_start(c0 + 1, 1)
        g_wait(0)
        s_start(c0, 0)
        return carry

    lax.fori_loop(0, (_NCH - 2) // 2, body, 0)
    g_wait(1)
    s_start(_NCH - 1, 1)
    s_wait(0)
    s_wait(1)


_CP = C // 2


@functools.lru_cache(maxsize=1)
def _sc_gather_kernel():
    mesh = plsc.VectorSubcoreMesh(core_axis_name="c", subcore_axis_name="s")
    return pl.kernel(
        _sc_gather_body,
        mesh=mesh,
        out_type=jax.ShapeDtypeStruct((4 * E, _CP), jnp.uint32),
        scratch_types=[
            pltpu.VMEM((_NCH, _R), jnp.int32),
            pltpu.VMEM((_R, _CP), jnp.uint32),
            pltpu.VMEM((_R, _CP), jnp.uint32),
            pltpu.SemaphoreType.DMA,
            pltpu.SemaphoreType.DMA,
            pltpu.SemaphoreType.DMA,
            pltpu.SemaphoreType.DMA,
        ],
    )


def _sc_gather(table, idxf):
    return _sc_gather_kernel()(table, idxf.reshape(_NW, _NCH, _R))


def _bf16_bits(xf32):
    xb = lax.bitcast_convert_type(xf32, jnp.uint32)
    r = xb + jnp.uint32(0x7FFF) + ((xb >> 16) & jnp.uint32(1))
    return r >> 16


def _pack_row(xf32):
    top = _bf16_bits(xf32[:, :_CP])
    bot = _bf16_bits(xf32[:, _CP:])
    return (bot << 16) | top


def _unpack_row(p):
    top = lax.bitcast_convert_type(p << 16, jnp.float32)
    bot = lax.bitcast_convert_type((p >> 16) << 16, jnp.float32)
    return jnp.concatenate([top, bot], axis=1)


def _tr_body(x_ref, o_ref):
    o_ref[:] = _pack_row(x_ref[:].T)


def _transpose_tc(x2d):
    return pl.pallas_call(
        _tr_body,
        grid=(E // _BT,),
        in_specs=[pl.BlockSpec((C, _BT), lambda i: (0, i))],
        out_specs=pl.BlockSpec((_BT, _CP), lambda i: (i, 0)),
        out_shape=jax.ShapeDtypeStruct((E, _CP), jnp.uint32),
    )(x2d)


def _combine_dot(nb_ref, w_ref, acc):
    n0 = _unpack_row(nb_ref[0])
    n1 = _unpack_row(nb_ref[1])
    n2 = _unpack_row(nb_ref[2])
    n3 = _unpack_row(nb_ref[3])
    acc = acc + jnp.dot(n0 + n2, w_ref[1], preferred_element_type=jnp.float32)
    acc = acc + jnp.dot(n1 + n3, w_ref[2], preferred_element_type=jnp.float32)
    acc = acc + jnp.dot(jnp.abs(n0 - n2), w_ref[3],
                        preferred_element_type=jnp.float32)
    acc = acc + jnp.dot(jnp.abs(n1 - n3), w_ref[4],
                        preferred_element_type=jnp.float32)
    return acc


def _mm1_body(x_ref, nb_ref, w_ref, b_ref, o_ref, obf_ref):
    acc = lax.dot_general(x_ref[:], w_ref[0], (((0,), (0,)), ((), ())),
                          preferred_element_type=jnp.float32)
    acc = _combine_dot(nb_ref, w_ref, acc) + b_ref[:]
    o_ref[:] = acc
    obf_ref[:] = _pack_row(acc)


def _mm2_body(x_ref, nb_ref, w_ref, b_ref, o_ref):
    acc = jnp.dot(x_ref[:], w_ref[0], preferred_element_type=jnp.float32)
    acc = _combine_dot(nb_ref, w_ref, acc) + b_ref[:]
    o_ref[:] = acc.T


def _mesh_conv1(x2d, nb, wt, b2d):
    return pl.pallas_call(
        _mm1_body,
        grid=(E // _BE,),
        in_specs=[
            pl.BlockSpec((C, _BE), lambda i: (0, i)),
            pl.BlockSpec((4, _BE, _CP), lambda i: (0, i, 0)),
            pl.BlockSpec((5, C, C), lambda i: (0, 0, 0)),
            pl.BlockSpec((1, C), lambda i: (0, 0)),
        ],
        out_specs=[
            pl.BlockSpec((_BE, C), lambda i: (i, 0)),
            pl.BlockSpec((_BE, _CP), lambda i: (i, 0)),
        ],
        out_shape=[
            jax.ShapeDtypeStruct((E, C), jnp.float32),
            jax.ShapeDtypeStruct((E, _CP), jnp.uint32),
        ],
    )(x2d, nb.reshape(4, E, _CP), wt, b2d)


def _mesh_conv2(h1, nb, wt, b2d):
    return pl.pallas_call(
        _mm2_body,
        grid=(E // _BE,),
        in_specs=[
            pl.BlockSpec((_BE, C), lambda i: (i, 0)),
            pl.BlockSpec((4, _BE, _CP), lambda i: (0, i, 0)),
            pl.BlockSpec((5, C, C), lambda i: (0, 0, 0)),
            pl.BlockSpec((1, C), lambda i: (0, 0)),
        ],
        out_specs=pl.BlockSpec((C, _BE), lambda i: (0, i)),
        out_shape=jax.ShapeDtypeStruct((C, E), jnp.float32),
    )(h1, nb.reshape(4, E, _CP), wt, b2d)


def kernel(input, meshes, W1, b1, W2, b2):
    x2d = input.reshape(C, E)
    idxf = meshes.reshape(E, 4).astype(jnp.int32).T.reshape(4 * E)
    wt1 = jnp.transpose(W1, (2, 1, 0))
    wt2 = jnp.transpose(W2, (2, 1, 0))
    xt_p = _transpose_tc(x2d)
    nb1 = _sc_gather(xt_p, idxf)
    h1, h1_p = _mesh_conv1(x2d, nb1, wt1, b1.reshape(1, C))
    nb2 = _sc_gather(h1_p, idxf)
    out_t = _mesh_conv2(h1, nb2, wt2, b2.reshape(1, C))
    return out_t.reshape(1, C, E)


# speedup vs baseline: 5.5061x; 1.0939x over previous
import functools

import jax
import jax.numpy as jnp
from jax import lax
from jax.experimental import pallas as pl
from jax.experimental.pallas import tpu as pltpu
from jax.experimental.pallas import tpu_sc as plsc

E = 160000
C = 256
_NW = 32
_R = 80
_BE = 640
_BT = 640

_NCH = (4 * E) // _NW // _R


_NB = 5
_LEAD = 2


def _sc_gather_body(table_hbm, idx_hbm, out_hbm, idx_all,
                    rows0, rows1, rows2, rows3, rows4,
                    g0, g1, g2, g3, g4, s0, s1, s2, s3, s4):
    wid = lax.axis_index("s") * 2 + lax.axis_index("c")
    nrows = (4 * E) // _NW
    base0 = wid * nrows
    pltpu.sync_copy(idx_hbm.at[wid], idx_all)

    rows = (rows0, rows1, rows2, rows3, rows4)
    gs = (g0, g1, g2, g3, g4)
    ss = (s0, s1, s2, s3, s4)

    def g_start(ci, b):
        pltpu.async_copy(table_hbm.at[idx_all.at[ci]], rows[b], gs[b])

    def g_wait(b):
        pltpu.make_async_copy(table_hbm.at[idx_all.at[0]], rows[b],
                              gs[b]).wait()

    def s_start(ci, b):
        pltpu.async_copy(rows[b], out_hbm.at[pl.ds(base0 + ci * _R, _R)],
                         ss[b])

    def s_wait(b):
        pltpu.make_async_copy(rows[b], out_hbm.at[pl.ds(base0, _R)],
                              ss[b]).wait()

    g_start(0, 0)
    g_start(1, 1)
    g_start(2, 2)
    g_wait(0)
    s_start(0, 0)
    g_start(3, 3)
    g_wait(1)
    s_start(1, 1)
    g_start(4, 4)
    g_wait(2)
    s_start(2, 2)
    s_wait(0)
    g_start(5, 0)
    g_wait(3)
    s_start(3, 3)
    s_wait(1)
    g_start(6, 1)
    g_wait(4)
    s_start(4, 4)

    def body(j, carry):
        c_base = _NB * j

        def step(b):
            ci = c_base + b
            bp = (b + _LEAD) % _NB
            s_wait(bp)
            ci2 = ci + _LEAD
            g_start(jnp.where(ci2 < _NCH, ci2, 0), bp)
            g_wait(b)
            s_start(ci, b)

        for b in range(_NB):
            step(b)
        return carry

    lax.fori_loop(1, _NCH // _NB, body, 0)
    g_wait(0)
    g_wait(1)
    for b in range(_LEAD, _NB):
        s_wait(b)


_CP = C // 2


@functools.lru_cache(maxsize=1)
def _sc_gather_kernel():
    mesh = plsc.VectorSubcoreMesh(core_axis_name="c", subcore_axis_name="s")
    return pl.kernel(
        _sc_gather_body,
        mesh=mesh,
        out_type=jax.ShapeDtypeStruct((4 * E, _CP), jnp.uint32),
        scratch_types=(
            [pltpu.VMEM((_NCH, _R), jnp.int32)]
            + [pltpu.VMEM((_R, _CP), jnp.uint32) for _ in range(_NB)]
            + [pltpu.SemaphoreType.DMA for _ in range(2 * _NB)]
        ),
    )


def _sc_gather(table, idxf):
    return _sc_gather_kernel()(table, idxf.reshape(_NW, _NCH, _R))


def _bf16_bits(xf32):
    xb = lax.bitcast_convert_type(xf32, jnp.uint32)
    r = xb + jnp.uint32(0x7FFF) + ((xb >> 16) & jnp.uint32(1))
    return r >> 16


def _pack_row(xf32):
    top = _bf16_bits(xf32[:, :_CP])
    bot = _bf16_bits(xf32[:, _CP:])
    return (bot << 16) | top


def _unpack_row(p):
    top = lax.bitcast_convert_type(p << 16, jnp.float32)
    bot = lax.bitcast_convert_type((p >> 16) << 16, jnp.float32)
    return jnp.concatenate([top, bot], axis=1)


def _tr_body(x_ref, o_ref):
    o_ref[:] = _pack_row(x_ref[:].T)


def _transpose_tc(x2d):
    return pl.pallas_call(
        _tr_body,
        grid=(E // _BT,),
        in_specs=[pl.BlockSpec((C, _BT), lambda i: (0, i))],
        out_specs=pl.BlockSpec((_BT, _CP), lambda i: (i, 0)),
        out_shape=jax.ShapeDtypeStruct((E, _CP), jnp.uint32),
    )(x2d)


def _combine_dot(nb_ref, w_ref, acc):
    n0 = _unpack_row(nb_ref[0])
    n1 = _unpack_row(nb_ref[1])
    n2 = _unpack_row(nb_ref[2])
    n3 = _unpack_row(nb_ref[3])
    acc = acc + jnp.dot(n0 + n2, w_ref[1], preferred_element_type=jnp.float32)
    acc = acc + jnp.dot(n1 + n3, w_ref[2], preferred_element_type=jnp.float32)
    acc = acc + jnp.dot(jnp.abs(n0 - n2), w_ref[3],
                        preferred_element_type=jnp.float32)
    acc = acc + jnp.dot(jnp.abs(n1 - n3), w_ref[4],
                        preferred_element_type=jnp.float32)
    return acc


def _mm_body(x_ref, nb_ref, w_ref, b_ref, o_ref, *, final):
    f0 = _unpack_row(x_ref[:])
    acc = jnp.dot(f0, w_ref[0], preferred_element_type=jnp.float32)
    acc = _combine_dot(nb_ref, w_ref, acc) + b_ref[:]
    o_ref[:] = acc.T if final else _pack_row(acc)


def _mesh_conv_tc(xt_p, nb, wt, b2d, final):
    if final:
        out_shape = jax.ShapeDtypeStruct((C, E), jnp.float32)
        out_spec = pl.BlockSpec((C, _BE), lambda i: (0, i))
    else:
        out_shape = jax.ShapeDtypeStruct((E, _CP), jnp.uint32)
        out_spec = pl.BlockSpec((_BE, _CP), lambda i: (i, 0))
    return pl.pallas_call(
        functools.partial(_mm_body, final=final),
        grid=(E // _BE,),
        in_specs=[
            pl.BlockSpec((_BE, _CP), lambda i: (i, 0)),
            pl.BlockSpec((4, _BE, _CP), lambda i: (0, i, 0)),
            pl.BlockSpec((5, C, C), lambda i: (0, 0, 0)),
            pl.BlockSpec((1, C), lambda i: (0, 0)),
        ],
        out_specs=out_spec,
        out_shape=out_shape,
    )(xt_p, nb.reshape(4, E, _CP), wt, b2d)


def kernel(input, meshes, W1, b1, W2, b2):
    x2d = input.reshape(C, E)
    idxf = meshes.reshape(E, 4).astype(jnp.int32).T.reshape(4 * E)
    wt1 = jnp.transpose(W1, (2, 1, 0))
    wt2 = jnp.transpose(W2, (2, 1, 0))
    xt_p = _transpose_tc(x2d)
    nb1 = _sc_gather(xt_p, idxf)
    h1_p = _mesh_conv_tc(xt_p, nb1, wt1, b1.reshape(1, C), False)
    nb2 = _sc_gather(h1_p, idxf)
    out_t = _mesh_conv_tc(h1_p, nb2, wt2, b2.reshape(1, C), True)
    return out_t.reshape(1, C, E)


# speedup vs baseline: 5.8642x; 1.0650x over previous
import functools

import jax
import jax.numpy as jnp
from jax import lax
from jax.experimental import pallas as pl
from jax.experimental.pallas import tpu as pltpu
from jax.experimental.pallas import tpu_sc as plsc

E = 160000
C = 256
_NW = 32
_R = 80
_BE = 640
_BT = 640

_NCH = (4 * E) // _NW // _R


_NB = 5
_LEAD = 2


def _sc_gather_body(table_hbm, idx_hbm, out_hbm, idx_all,
                    rows0, rows1, rows2, rows3, rows4,
                    g0, g1, g2, g3, g4, s0, s1, s2, s3, s4):
    nch = idx_all.shape[0]
    wid = lax.axis_index("s") * 2 + lax.axis_index("c")
    nrows = out_hbm.shape[0] // _NW
    base0 = wid * nrows
    pltpu.sync_copy(idx_hbm.at[wid], idx_all)

    rows = (rows0, rows1, rows2, rows3, rows4)
    gs = (g0, g1, g2, g3, g4)
    ss = (s0, s1, s2, s3, s4)

    def g_start(ci, b):
        pltpu.async_copy(table_hbm.at[idx_all.at[ci]], rows[b], gs[b])

    def g_wait(b):
        pltpu.make_async_copy(table_hbm.at[idx_all.at[0]], rows[b],
                              gs[b]).wait()

    def s_start(ci, b):
        pltpu.async_copy(rows[b], out_hbm.at[pl.ds(base0 + ci * _R, _R)],
                         ss[b])

    def s_wait(b):
        pltpu.make_async_copy(rows[b], out_hbm.at[pl.ds(base0, _R)],
                              ss[b]).wait()

    g_start(0, 0)
    g_start(1, 1)
    g_start(2, 2)
    g_wait(0)
    s_start(0, 0)
    g_start(3, 3)
    g_wait(1)
    s_start(1, 1)
    g_start(4, 4)
    g_wait(2)
    s_start(2, 2)
    s_wait(0)
    g_start(5, 0)
    g_wait(3)
    s_start(3, 3)
    s_wait(1)
    g_start(6, 1)
    g_wait(4)
    s_start(4, 4)

    def body(j, carry):
        c_base = _NB * j

        def step(b):
            ci = c_base + b
            bp = (b + _LEAD) % _NB
            s_wait(bp)
            ci2 = ci + _LEAD
            g_start(jnp.where(ci2 < nch, ci2, 0), bp)
            g_wait(b)
            s_start(ci, b)

        for b in range(_NB):
            step(b)
        return carry

    lax.fori_loop(1, nch // _NB, body, 0)
    g_wait(0)
    g_wait(1)
    for b in range(_LEAD, _NB):
        s_wait(b)


_CP = C // 2


@functools.lru_cache(maxsize=2)
def _sc_gather_kernel(rows_total):
    nch = rows_total // _NW // _R
    mesh = plsc.VectorSubcoreMesh(core_axis_name="c", subcore_axis_name="s")
    return pl.kernel(
        _sc_gather_body,
        mesh=mesh,
        out_type=jax.ShapeDtypeStruct((rows_total, _CP), jnp.uint32),
        scratch_types=(
            [pltpu.VMEM((nch, _R), jnp.int32)]
            + [pltpu.VMEM((_R, _CP), jnp.uint32) for _ in range(_NB)]
            + [pltpu.SemaphoreType.DMA for _ in range(2 * _NB)]
        ),
    )


def _sc_gather(table, idxf):
    rows_total = idxf.shape[0]
    nch = rows_total // _NW // _R
    return _sc_gather_kernel(rows_total)(table, idxf.reshape(_NW, nch, _R))


def _bf16_bits(xf32):
    xb = lax.bitcast_convert_type(xf32, jnp.uint32)
    r = xb + jnp.uint32(0x7FFF) + ((xb >> 16) & jnp.uint32(1))
    return r >> 16


def _pack_row(xf32):
    top = _bf16_bits(xf32[:, :_CP])
    bot = _bf16_bits(xf32[:, _CP:])
    return (bot << 16) | top


def _unpack_row(p):
    top = lax.bitcast_convert_type(p << 16, jnp.float32)
    bot = lax.bitcast_convert_type((p >> 16) << 16, jnp.float32)
    return jnp.concatenate([top, bot], axis=1)


def _tr_body(x_ref, o_ref):
    o_ref[:] = _pack_row(x_ref[:].T)


def _transpose_tc(x2d):
    return pl.pallas_call(
        _tr_body,
        grid=(E // _BT,),
        in_specs=[pl.BlockSpec((C, _BT), lambda i: (0, i))],
        out_specs=pl.BlockSpec((_BT, _CP), lambda i: (i, 0)),
        out_shape=jax.ShapeDtypeStruct((E, _CP), jnp.uint32),
    )(x2d)


def _combine_dot(nb_ref, w_ref, acc):
    n0 = _unpack_row(nb_ref[0])
    n1 = _unpack_row(nb_ref[1])
    n2 = _unpack_row(nb_ref[2])
    n3 = _unpack_row(nb_ref[3])
    acc = acc + jnp.dot(n0 + n2, w_ref[1], preferred_element_type=jnp.float32)
    acc = acc + jnp.dot(n1 + n3, w_ref[2], preferred_element_type=jnp.float32)
    acc = acc + jnp.dot(jnp.abs(n0 - n2), w_ref[3],
                        preferred_element_type=jnp.float32)
    acc = acc + jnp.dot(jnp.abs(n1 - n3), w_ref[4],
                        preferred_element_type=jnp.float32)
    return acc


def _mm_body(x_ref, nb_ref, w_ref, b_ref, o_ref, *, final):
    f0 = _unpack_row(x_ref[:])
    acc = jnp.dot(f0, w_ref[0], preferred_element_type=jnp.float32)
    acc = _combine_dot(nb_ref, w_ref, acc) + b_ref[:]
    o_ref[:] = acc.T if final else _pack_row(acc)


def _mm_half_body(prev_ref, x_ref, nb_ref, w_ref, b_ref, o_ref, *, final):
    del prev_ref
    _mm_body(x_ref, nb_ref, w_ref, b_ref, o_ref, final=final)


_EH = E // 2


def _mesh_conv_half(prev, xt_p, nbh, wt, b2d, final, half):
    nblk = _EH // _BE
    off = half * nblk
    if final:
        out_shape = jax.ShapeDtypeStruct((C, E), jnp.float32)
        out_spec = pl.BlockSpec((C, _BE), lambda i: (0, i + off))
    else:
        out_shape = jax.ShapeDtypeStruct((E, _CP), jnp.uint32)
        out_spec = pl.BlockSpec((_BE, _CP), lambda i: (i + off, 0))
    body = functools.partial(_mm_half_body if prev is not None else _mm_body,
                             final=final)
    in_specs = [
        pl.BlockSpec((_BE, _CP), lambda i: (i + off, 0)),
        pl.BlockSpec((4, _BE, _CP), lambda i: (0, i, 0)),
        pl.BlockSpec((5, C, C), lambda i: (0, 0, 0)),
        pl.BlockSpec((1, C), lambda i: (0, 0)),
    ]
    args = (xt_p, nbh.reshape(4, _EH, _CP), wt, b2d)
    alias = {}
    if prev is not None:
        in_specs = [pl.BlockSpec(memory_space=pl.ANY)] + in_specs
        args = (prev,) + args
        alias = {0: 0}
    return pl.pallas_call(
        body,
        grid=(nblk,),
        in_specs=in_specs,
        out_specs=out_spec,
        out_shape=out_shape,
        input_output_aliases=alias,
    )(*args)


def kernel(input, meshes, W1, b1, W2, b2):
    x2d = input.reshape(C, E)
    m = meshes.reshape(E, 4).astype(jnp.int32)
    idx_a = m[:_EH].T.reshape(2 * E)
    idx_b = m[_EH:].T.reshape(2 * E)
    wt1 = jnp.transpose(W1, (2, 1, 0))
    wt2 = jnp.transpose(W2, (2, 1, 0))
    b1r = b1.reshape(1, C)
    b2r = b2.reshape(1, C)
    xt_p = _transpose_tc(x2d)
    nb1a = _sc_gather(xt_p, idx_a)
    nb1b = _sc_gather(xt_p, idx_b)
    h1pa = _mesh_conv_half(None, xt_p, nb1a, wt1, b1r, False, 0)
    h1p = _mesh_conv_half(h1pa, xt_p, nb1b, wt1, b1r, False, 1)
    nb2a = _sc_gather(h1p, idx_a)
    nb2b = _sc_gather(h1p, idx_b)
    outa = _mesh_conv_half(None, h1p, nb2a, wt2, b2r, True, 0)
    out_t = _mesh_conv_half(outa, h1p, nb2b, wt2, b2r, True, 1)
    return out_t.reshape(1, C, E)
